# baseline (speedup 1.0000x reference)
import functools

import jax
import jax.numpy as jnp
from jax import lax
from jax.experimental import pallas as pl
from jax.experimental.pallas import tpu as pltpu
from jax.experimental.pallas import tpu_sc as plsc

N = 2048
D = 768
F = 768
E = 64
TILE = 256

_NC = 2
_NS = 16
_NW = _NC * _NS
_ROWS_PER_WORKER = N // _NW


def _router_body(x_ref, wg_ref, bg_ref, dest_ref, s_ref, e_ref):
    f32 = jnp.float32
    logits = jnp.dot(x_ref[:], wg_ref[:], preferred_element_type=f32)
    logits = logits + bg_ref[:]
    m = jnp.max(logits, axis=1, keepdims=True)
    col = lax.broadcasted_iota(jnp.int32, (N, E), 1)
    idx = jnp.min(jnp.where(logits == m, col, jnp.int32(E)),
                  axis=1, keepdims=True)
    hf = (col == idx).astype(f32)

    rown = lax.broadcasted_iota(jnp.int32, (N, N), 0)
    coln = lax.broadcasted_iota(jnp.int32, (N, N), 1)
    lstrict = (coln < rown).astype(jnp.bfloat16)
    lh = jnp.dot(lstrict, hf.astype(jnp.bfloat16), preferred_element_type=f32)
    rank = jnp.sum(lh * hf, axis=1, keepdims=True)

    counts_row = jnp.sum(hf, axis=0, keepdims=True)
    re_ = lax.broadcasted_iota(jnp.int32, (E, E), 0)
    ce_ = lax.broadcasted_iota(jnp.int32, (E, E), 1)
    counts_b = jnp.broadcast_to(counts_row, (E, E))
    starts_col = jnp.sum(jnp.where(ce_ < re_, counts_b, 0.0),
                         axis=1, keepdims=True)
    counts_col = jnp.sum(jnp.where(ce_ == re_, counts_b, 0.0),
                         axis=1, keepdims=True)
    cum_col = starts_col + counts_col
    starts_row = jnp.sum(jnp.where(ce_ == re_,
                                   jnp.broadcast_to(starts_col, (E, E)), 0.0),
                         axis=0, keepdims=True)

    dest = rank + jnp.sum(starts_row * hf, axis=1, keepdims=True)
    dest_ref[:] = dest.astype(jnp.int32)
    s_ref[:] = starts_col.astype(jnp.int32)
    e_ref[:] = cum_col.astype(jnp.int32)


def _route(x_flat, Wg, bg):
    i32 = jnp.int32
    dest, starts, ends = pl.pallas_call(
        _router_body,
        out_shape=(
            jax.ShapeDtypeStruct((N, 1), i32),
            jax.ShapeDtypeStruct((E, 1), i32),
            jax.ShapeDtypeStruct((E, 1), i32),
        ),
    )(x_flat, Wg, bg.reshape(1, E))
    return dest.reshape(-1), starts.reshape(-1), ends.reshape(-1)


def _scatter_rows(src, indices):
    mesh = plsc.VectorSubcoreMesh(core_axis_name="c", subcore_axis_name="s")

    @functools.partial(
        pl.kernel,
        out_type=jax.ShapeDtypeStruct((N, D), jnp.float32),
        mesh=mesh,
        scratch_types=[
            pltpu.VMEM((_ROWS_PER_WORKER,), jnp.int32),
            pltpu.VMEM((_ROWS_PER_WORKER, D), jnp.float32),
            pltpu.SemaphoreType.DMA,
        ],
    )
    def k(src_hbm, idx_hbm, out_hbm, idx_v, rows_v, sem):
        wid = lax.axis_index("s") * _NC + lax.axis_index("c")
        base = wid * _ROWS_PER_WORKER
        pltpu.sync_copy(idx_hbm.at[pl.ds(base, _ROWS_PER_WORKER)], idx_v)
        pltpu.sync_copy(src_hbm.at[pl.ds(base, _ROWS_PER_WORKER)], rows_v)
        pltpu.async_copy(rows_v, out_hbm.at[idx_v], sem).wait()

    return k(src, indices)


def _gather_rows(table, indices):
    mesh = plsc.VectorSubcoreMesh(core_axis_name="c", subcore_axis_name="s")

    @functools.partial(
        pl.kernel,
        out_type=jax.ShapeDtypeStruct((N, D), jnp.float32),
        mesh=mesh,
        scratch_types=[
            pltpu.VMEM((_ROWS_PER_WORKER,), jnp.int32),
            pltpu.VMEM((_ROWS_PER_WORKER, D), jnp.float32),
            pltpu.SemaphoreType.DMA,
        ],
    )
    def k(table_hbm, idx_hbm, out_hbm, idx_v, rows_v, sem):
        wid = lax.axis_index("s") * _NC + lax.axis_index("c")
        base = wid * _ROWS_PER_WORKER
        pltpu.sync_copy(idx_hbm.at[pl.ds(base, _ROWS_PER_WORKER)], idx_v)
        pltpu.async_copy(table_hbm.at[idx_v], rows_v, sem).wait()
        pltpu.sync_copy(rows_v, out_hbm.at[pl.ds(base, _ROWS_PER_WORKER)])

    return k(table, indices)


_NBUF = 4


def _ffn_body(s_ref, e_ref, x_ref, w1_hbm, b1_ref, w2_hbm, b2_ref, o_ref,
              w1_buf, w2_buf, sem1, sem2):
    ei = pl.program_id(0)

    def w1_copy(j, slot):
        return pltpu.make_async_copy(w1_hbm.at[j], w1_buf.at[slot],
                                     sem1.at[slot])

    def w2_copy(j, slot):
        return pltpu.make_async_copy(w2_hbm.at[j], w2_buf.at[slot],
                                     sem2.at[slot])

    @pl.when(ei == 0)
    def _():
        o_ref[:] = jnp.zeros_like(o_ref)
        for j in range(_NBUF - 1):
            w1_copy(j, j).start()
            w2_copy(j, j).start()

    nxt = ei + _NBUF - 1

    @pl.when(nxt < E)
    def _():
        nslot = lax.rem(nxt, _NBUF)
        w1_copy(nxt, nslot).start()
        w2_copy(nxt, nslot).start()

    slot = lax.rem(ei, _NBUF)
    w1_copy(ei, slot).wait()
    w2_copy(ei, slot).wait()

    start = s_ref[ei]
    end = e_ref[ei]
    astart = (start // 8) * 8
    nblk = (end - astart + TILE - 1) // TILE
    w1 = w1_buf[slot].astype(jnp.bfloat16)
    w2 = w2_buf[slot].astype(jnp.bfloat16)
    b1 = b1_ref[ei]
    b2 = b2_ref[ei]

    def body(k, carry):
        bstart = jnp.minimum(astart + k * TILE, N - TILE)
        xb = x_ref[pl.ds(bstart, TILE), :].astype(jnp.bfloat16)
        h = jnp.dot(xb, w1, preferred_element_type=jnp.float32) + b1
        h = 0.5 * h * (1.0 + lax.erf(h * 0.7071067811865476))
        y = jnp.dot(h.astype(jnp.bfloat16), w2,
                    preferred_element_type=jnp.float32) + b2
        row = bstart + lax.broadcasted_iota(jnp.int32, (TILE, 1), 0)
        mask = (row >= start) & (row < end)
        o_ref[pl.ds(bstart, TILE), :] = jnp.where(
            mask, y, o_ref[pl.ds(bstart, TILE), :])
        return carry

    lax.fori_loop(0, nblk, body, 0)


def _grouped_ffn(x_sorted, W1, b1, W2, b2, starts, ends):
    grid_spec = pltpu.PrefetchScalarGridSpec(
        num_scalar_prefetch=2,
        grid=(E,),
        in_specs=[
            pl.BlockSpec((N, D), lambda e, s, t: (0, 0)),
            pl.BlockSpec(memory_space=pl.ANY),
            pl.BlockSpec((E, 1, F), lambda e, s, t: (0, 0, 0)),
            pl.BlockSpec(memory_space=pl.ANY),
            pl.BlockSpec((E, 1, D), lambda e, s, t: (0, 0, 0)),
        ],
        out_specs=pl.BlockSpec((N, D), lambda e, s, t: (0, 0)),
        scratch_shapes=[
            pltpu.VMEM((_NBUF, D, F), jnp.float32),
            pltpu.VMEM((_NBUF, F, D), jnp.float32),
            pltpu.SemaphoreType.DMA((_NBUF,)),
            pltpu.SemaphoreType.DMA((_NBUF,)),
        ],
    )
    return pl.pallas_call(
        _ffn_body,
        grid_spec=grid_spec,
        out_shape=jax.ShapeDtypeStruct((N, D), jnp.float32),
        compiler_params=pltpu.CompilerParams(
            dimension_semantics=("arbitrary",),
        ),
    )(starts, ends, x_sorted,
      W1, b1.reshape(E, 1, F), W2, b2.reshape(E, 1, D))


def kernel(x, Wg, bg, W1, b1, W2, b2):
    B, T, _ = x.shape
    x_flat = x.reshape(N, D)
    dest, starts, ends = _route(x_flat, Wg, bg)
    x_sorted = _scatter_rows(x_flat, dest)
    out_sorted = _grouped_ffn(x_sorted, W1, b1, W2, b2, starts, ends)
    out = _gather_rows(out_sorted, dest)
    return out.reshape(B, T, D)


# speedup vs baseline: 1.0011x; 1.0011x over previous
import functools

import jax
import jax.numpy as jnp
from jax import lax
from jax.experimental import pallas as pl
from jax.experimental.pallas import tpu as pltpu
from jax.experimental.pallas import tpu_sc as plsc

N = 2048
D = 768
F = 768
E = 64
TILE = 256

_NC = 2
_NS = 16
_NW = _NC * _NS
_ROWS_PER_WORKER = N // _NW


def _router_body(x_ref, wg_ref, bg_ref, dest_ref, s_ref, e_ref):
    f32 = jnp.float32
    logits = jnp.dot(x_ref[:], wg_ref[:], preferred_element_type=f32)
    logits = logits + bg_ref[:]
    m = jnp.max(logits, axis=1, keepdims=True)
    col = lax.broadcasted_iota(jnp.int32, (N, E), 1)
    idx = jnp.min(jnp.where(logits == m, col, jnp.int32(E)),
                  axis=1, keepdims=True)
    hf = (col == idx).astype(f32)

    rown = lax.broadcasted_iota(jnp.int32, (N, N), 0)
    coln = lax.broadcasted_iota(jnp.int32, (N, N), 1)
    lstrict = (coln < rown).astype(jnp.bfloat16)
    lh = jnp.dot(lstrict, hf.astype(jnp.bfloat16), preferred_element_type=f32)
    rank = jnp.sum(lh * hf, axis=1, keepdims=True)

    counts_row = jnp.sum(hf, axis=0, keepdims=True)
    re_ = lax.broadcasted_iota(jnp.int32, (E, E), 0)
    ce_ = lax.broadcasted_iota(jnp.int32, (E, E), 1)
    counts_b = jnp.broadcast_to(counts_row, (E, E))
    starts_col = jnp.sum(jnp.where(ce_ < re_, counts_b, 0.0),
                         axis=1, keepdims=True)
    counts_col = jnp.sum(jnp.where(ce_ == re_, counts_b, 0.0),
                         axis=1, keepdims=True)
    cum_col = starts_col + counts_col
    starts_row = jnp.sum(jnp.where(ce_ == re_,
                                   jnp.broadcast_to(starts_col, (E, E)), 0.0),
                         axis=0, keepdims=True)

    dest = rank + jnp.sum(starts_row * hf, axis=1, keepdims=True)
    dest_ref[:] = dest.astype(jnp.int32)
    s_ref[:] = starts_col.astype(jnp.int32)
    e_ref[:] = cum_col.astype(jnp.int32)


def _route(x_flat, Wg, bg):
    i32 = jnp.int32
    dest, starts, ends = pl.pallas_call(
        _router_body,
        out_shape=(
            jax.ShapeDtypeStruct((N, 1), i32),
            jax.ShapeDtypeStruct((E, 1), i32),
            jax.ShapeDtypeStruct((E, 1), i32),
        ),
    )(x_flat, Wg, bg.reshape(1, E))
    return dest.reshape(-1), starts.reshape(-1), ends.reshape(-1)


def _scatter_rows(src, indices):
    mesh = plsc.VectorSubcoreMesh(core_axis_name="c", subcore_axis_name="s")

    @functools.partial(
        pl.kernel,
        out_type=jax.ShapeDtypeStruct((N, D), jnp.float32),
        mesh=mesh,
        scratch_types=[
            pltpu.VMEM((_ROWS_PER_WORKER,), jnp.int32),
            pltpu.VMEM((_ROWS_PER_WORKER, D), jnp.float32),
            pltpu.SemaphoreType.DMA,
        ],
    )
    def k(src_hbm, idx_hbm, out_hbm, idx_v, rows_v, sem):
        wid = lax.axis_index("s") * _NC + lax.axis_index("c")
        base = wid * _ROWS_PER_WORKER
        pltpu.sync_copy(idx_hbm.at[pl.ds(base, _ROWS_PER_WORKER)], idx_v)
        pltpu.sync_copy(src_hbm.at[pl.ds(base, _ROWS_PER_WORKER)], rows_v)
        pltpu.async_copy(rows_v, out_hbm.at[idx_v], sem).wait()

    return k(src, indices)


def _gather_rows(table, indices):
    mesh = plsc.VectorSubcoreMesh(core_axis_name="c", subcore_axis_name="s")

    @functools.partial(
        pl.kernel,
        out_type=jax.ShapeDtypeStruct((N, D), jnp.float32),
        mesh=mesh,
        scratch_types=[
            pltpu.VMEM((_ROWS_PER_WORKER,), jnp.int32),
            pltpu.VMEM((_ROWS_PER_WORKER, D), jnp.float32),
            pltpu.SemaphoreType.DMA,
        ],
    )
    def k(table_hbm, idx_hbm, out_hbm, idx_v, rows_v, sem):
        wid = lax.axis_index("s") * _NC + lax.axis_index("c")
        base = wid * _ROWS_PER_WORKER
        pltpu.sync_copy(idx_hbm.at[pl.ds(base, _ROWS_PER_WORKER)], idx_v)
        pltpu.async_copy(table_hbm.at[idx_v], rows_v, sem).wait()
        pltpu.sync_copy(rows_v, out_hbm.at[pl.ds(base, _ROWS_PER_WORKER)])

    return k(table, indices)


_NBUF = 3


def _ffn_body(s_ref, e_ref, x_ref, w1_hbm, b1_ref, w2_hbm, b2_ref, o_ref,
              w1_buf, w2_buf, sem1, sem2):
    ei = pl.program_id(0)

    def w1_copy(j, slot):
        return pltpu.make_async_copy(w1_hbm.at[j], w1_buf.at[slot],
                                     sem1.at[slot])

    def w2_copy(j, slot):
        return pltpu.make_async_copy(w2_hbm.at[j], w2_buf.at[slot],
                                     sem2.at[slot])

    @pl.when(ei == 0)
    def _():
        o_ref[:] = jnp.zeros_like(o_ref)
        for j in range(_NBUF - 1):
            w1_copy(j, j).start()
            w2_copy(j, j).start()

    nxt = ei + _NBUF - 1

    @pl.when(nxt < E)
    def _():
        nslot = lax.rem(nxt, _NBUF)
        w1_copy(nxt, nslot).start()
        w2_copy(nxt, nslot).start()

    slot = lax.rem(ei, _NBUF)
    w1_copy(ei, slot).wait()
    w2_copy(ei, slot).wait()

    start = s_ref[ei]
    end = e_ref[ei]
    astart = (start // 8) * 8
    nblk = (end - astart + TILE - 1) // TILE
    w1 = w1_buf[slot].astype(jnp.bfloat16)
    w2 = w2_buf[slot].astype(jnp.bfloat16)
    b1 = b1_ref[ei]
    b2 = b2_ref[ei]

    def body(k, carry):
        bstart = jnp.minimum(astart + k * TILE, N - TILE)
        xb = x_ref[pl.ds(bstart, TILE), :].astype(jnp.bfloat16)
        h = jnp.dot(xb, w1, preferred_element_type=jnp.float32) + b1
        h = 0.5 * h * (1.0 + lax.erf(h * 0.7071067811865476))
        y = jnp.dot(h.astype(jnp.bfloat16), w2,
                    preferred_element_type=jnp.float32) + b2
        row = bstart + lax.broadcasted_iota(jnp.int32, (TILE, 1), 0)
        mask = (row >= start) & (row < end)
        o_ref[pl.ds(bstart, TILE), :] = jnp.where(
            mask, y, o_ref[pl.ds(bstart, TILE), :])
        return carry

    lax.fori_loop(0, nblk, body, 0)


def _grouped_ffn(x_sorted, W1, b1, W2, b2, starts, ends):
    grid_spec = pltpu.PrefetchScalarGridSpec(
        num_scalar_prefetch=2,
        grid=(E,),
        in_specs=[
            pl.BlockSpec((N, D), lambda e, s, t: (0, 0)),
            pl.BlockSpec(memory_space=pl.ANY),
            pl.BlockSpec((E, 1, F), lambda e, s, t: (0, 0, 0)),
            pl.BlockSpec(memory_space=pl.ANY),
            pl.BlockSpec((E, 1, D), lambda e, s, t: (0, 0, 0)),
        ],
        out_specs=pl.BlockSpec((N, D), lambda e, s, t: (0, 0)),
        scratch_shapes=[
            pltpu.VMEM((_NBUF, D, F), jnp.float32),
            pltpu.VMEM((_NBUF, F, D), jnp.float32),
            pltpu.SemaphoreType.DMA((_NBUF,)),
            pltpu.SemaphoreType.DMA((_NBUF,)),
        ],
    )
    return pl.pallas_call(
        _ffn_body,
        grid_spec=grid_spec,
        out_shape=jax.ShapeDtypeStruct((N, D), jnp.float32),
        compiler_params=pltpu.CompilerParams(
            dimension_semantics=("arbitrary",),
        ),
    )(starts, ends, x_sorted,
      W1, b1.reshape(E, 1, F), W2, b2.reshape(E, 1, D))


def kernel(x, Wg, bg, W1, b1, W2, b2):
    B, T, _ = x.shape
    x_flat = x.reshape(N, D)
    dest, starts, ends = _route(x_flat, Wg, bg)
    x_sorted = _scatter_rows(x_flat, dest)
    out_sorted = _grouped_ffn(x_sorted, W1, b1, W2, b2, starts, ends)
    out = _gather_rows(out_sorted, dest)
    return out.reshape(B, T, D)


# speedup vs baseline: 1.0211x; 1.0200x over previous
import functools

import jax
import jax.numpy as jnp
from jax import lax
from jax.experimental import pallas as pl
from jax.experimental.pallas import tpu as pltpu
from jax.experimental.pallas import tpu_sc as plsc

N = 2048
D = 768
F = 768
E = 64
TILE = 256

_NC = 2
_NS = 16
_NW = _NC * _NS
_ROWS_PER_WORKER = N // _NW


def _router_body(x_ref, wg_ref, bg_ref, dest_ref, s_ref, e_ref):
    f32 = jnp.float32
    logits = jnp.dot(x_ref[:], wg_ref[:], preferred_element_type=f32)
    logits = logits + bg_ref[:]
    m = jnp.max(logits, axis=1, keepdims=True)
    col = lax.broadcasted_iota(jnp.int32, (N, E), 1)
    idx = jnp.min(jnp.where(logits == m, col, jnp.int32(E)),
                  axis=1, keepdims=True)
    hf = (col == idx).astype(f32)

    _C = 256
    rc = lax.broadcasted_iota(jnp.int32, (_C, _C), 0)
    cc = lax.broadcasted_iota(jnp.int32, (_C, _C), 1)
    l256 = (cc < rc).astype(jnp.bfloat16)
    prefix = jnp.zeros((1, E), f32)
    rank_chunks = []
    for t in range(N // _C):
        ht = hf[t * _C:(t + 1) * _C]
        lh_t = jnp.dot(l256, ht.astype(jnp.bfloat16),
                       preferred_element_type=f32) + prefix
        rank_chunks.append(jnp.sum(lh_t * ht, axis=1, keepdims=True))
        prefix = prefix + jnp.sum(ht, axis=0, keepdims=True)
    rank = jnp.concatenate(rank_chunks, axis=0)

    counts_row = jnp.sum(hf, axis=0, keepdims=True)
    re_ = lax.broadcasted_iota(jnp.int32, (E, E), 0)
    ce_ = lax.broadcasted_iota(jnp.int32, (E, E), 1)
    counts_b = jnp.broadcast_to(counts_row, (E, E))
    starts_col = jnp.sum(jnp.where(ce_ < re_, counts_b, 0.0),
                         axis=1, keepdims=True)
    counts_col = jnp.sum(jnp.where(ce_ == re_, counts_b, 0.0),
                         axis=1, keepdims=True)
    cum_col = starts_col + counts_col
    starts_row = jnp.sum(jnp.where(ce_ == re_,
                                   jnp.broadcast_to(starts_col, (E, E)), 0.0),
                         axis=0, keepdims=True)

    dest = rank + jnp.sum(starts_row * hf, axis=1, keepdims=True)
    dest_ref[:] = dest.astype(jnp.int32)
    s_ref[:] = starts_col.astype(jnp.int32)
    e_ref[:] = cum_col.astype(jnp.int32)


def _route(x_flat, Wg, bg):
    i32 = jnp.int32
    dest, starts, ends = pl.pallas_call(
        _router_body,
        out_shape=(
            jax.ShapeDtypeStruct((N, 1), i32),
            jax.ShapeDtypeStruct((E, 1), i32),
            jax.ShapeDtypeStruct((E, 1), i32),
        ),
    )(x_flat, Wg, bg.reshape(1, E))
    return dest.reshape(-1), starts.reshape(-1), ends.reshape(-1)


def _scatter_rows(src, indices):
    mesh = plsc.VectorSubcoreMesh(core_axis_name="c", subcore_axis_name="s")

    @functools.partial(
        pl.kernel,
        out_type=jax.ShapeDtypeStruct((N, D), jnp.float32),
        mesh=mesh,
        scratch_types=[
            pltpu.VMEM((_ROWS_PER_WORKER,), jnp.int32),
            pltpu.VMEM((_ROWS_PER_WORKER, D), jnp.float32),
            pltpu.SemaphoreType.DMA,
        ],
    )
    def k(src_hbm, idx_hbm, out_hbm, idx_v, rows_v, sem):
        wid = lax.axis_index("s") * _NC + lax.axis_index("c")
        base = wid * _ROWS_PER_WORKER
        pltpu.sync_copy(idx_hbm.at[pl.ds(base, _ROWS_PER_WORKER)], idx_v)
        pltpu.sync_copy(src_hbm.at[pl.ds(base, _ROWS_PER_WORKER)], rows_v)
        pltpu.async_copy(rows_v, out_hbm.at[idx_v], sem).wait()

    return k(src, indices)


def _gather_rows(table, indices):
    mesh = plsc.VectorSubcoreMesh(core_axis_name="c", subcore_axis_name="s")

    @functools.partial(
        pl.kernel,
        out_type=jax.ShapeDtypeStruct((N, D), jnp.float32),
        mesh=mesh,
        scratch_types=[
            pltpu.VMEM((_ROWS_PER_WORKER,), jnp.int32),
            pltpu.VMEM((_ROWS_PER_WORKER, D), jnp.float32),
            pltpu.SemaphoreType.DMA,
        ],
    )
    def k(table_hbm, idx_hbm, out_hbm, idx_v, rows_v, sem):
        wid = lax.axis_index("s") * _NC + lax.axis_index("c")
        base = wid * _ROWS_PER_WORKER
        pltpu.sync_copy(idx_hbm.at[pl.ds(base, _ROWS_PER_WORKER)], idx_v)
        pltpu.async_copy(table_hbm.at[idx_v], rows_v, sem).wait()
        pltpu.sync_copy(rows_v, out_hbm.at[pl.ds(base, _ROWS_PER_WORKER)])

    return k(table, indices)


_NBUF = 3


def _ffn_body(s_ref, e_ref, x_ref, w1_hbm, b1_ref, w2_hbm, b2_ref, o_ref,
              w1_buf, w2_buf, sem1, sem2):
    ei = pl.program_id(0)

    def w1_copy(j, slot):
        return pltpu.make_async_copy(w1_hbm.at[j], w1_buf.at[slot],
                                     sem1.at[slot])

    def w2_copy(j, slot):
        return pltpu.make_async_copy(w2_hbm.at[j], w2_buf.at[slot],
                                     sem2.at[slot])

    @pl.when(ei == 0)
    def _():
        o_ref[:] = jnp.zeros_like(o_ref)
        for j in range(_NBUF - 1):
            w1_copy(j, j).start()
            w2_copy(j, j).start()

    nxt = ei + _NBUF - 1

    @pl.when(nxt < E)
    def _():
        nslot = lax.rem(nxt, _NBUF)
        w1_copy(nxt, nslot).start()
        w2_copy(nxt, nslot).start()

    slot = lax.rem(ei, _NBUF)
    w1_copy(ei, slot).wait()
    w2_copy(ei, slot).wait()

    start = s_ref[ei]
    end = e_ref[ei]
    astart = (start // 8) * 8
    nblk = (end - astart + TILE - 1) // TILE
    w1 = w1_buf[slot].astype(jnp.bfloat16)
    w2 = w2_buf[slot].astype(jnp.bfloat16)
    b1 = b1_ref[ei]
    b2 = b2_ref[ei]

    def body(k, carry):
        bstart = jnp.minimum(astart + k * TILE, N - TILE)
        xb = x_ref[pl.ds(bstart, TILE), :].astype(jnp.bfloat16)
        h = jnp.dot(xb, w1, preferred_element_type=jnp.float32) + b1
        h = 0.5 * h * (1.0 + lax.erf(h * 0.7071067811865476))
        y = jnp.dot(h.astype(jnp.bfloat16), w2,
                    preferred_element_type=jnp.float32) + b2
        row = bstart + lax.broadcasted_iota(jnp.int32, (TILE, 1), 0)
        mask = (row >= start) & (row < end)
        o_ref[pl.ds(bstart, TILE), :] = jnp.where(
            mask, y, o_ref[pl.ds(bstart, TILE), :])
        return carry

    lax.fori_loop(0, nblk, body, 0)


def _grouped_ffn(x_sorted, W1, b1, W2, b2, starts, ends):
    grid_spec = pltpu.PrefetchScalarGridSpec(
        num_scalar_prefetch=2,
        grid=(E,),
        in_specs=[
            pl.BlockSpec((N, D), lambda e, s, t: (0, 0)),
            pl.BlockSpec(memory_space=pl.ANY),
            pl.BlockSpec((E, 1, F), lambda e, s, t: (0, 0, 0)),
            pl.BlockSpec(memory_space=pl.ANY),
            pl.BlockSpec((E, 1, D), lambda e, s, t: (0, 0, 0)),
        ],
        out_specs=pl.BlockSpec((N, D), lambda e, s, t: (0, 0)),
        scratch_shapes=[
            pltpu.VMEM((_NBUF, D, F), jnp.float32),
            pltpu.VMEM((_NBUF, F, D), jnp.float32),
            pltpu.SemaphoreType.DMA((_NBUF,)),
            pltpu.SemaphoreType.DMA((_NBUF,)),
        ],
    )
    return pl.pallas_call(
        _ffn_body,
        grid_spec=grid_spec,
        out_shape=jax.ShapeDtypeStruct((N, D), jnp.float32),
        compiler_params=pltpu.CompilerParams(
            dimension_semantics=("arbitrary",),
        ),
    )(starts, ends, x_sorted,
      W1, b1.reshape(E, 1, F), W2, b2.reshape(E, 1, D))


def kernel(x, Wg, bg, W1, b1, W2, b2):
    B, T, _ = x.shape
    x_flat = x.reshape(N, D)
    dest, starts, ends = _route(x_flat, Wg, bg)
    x_sorted = _scatter_rows(x_flat, dest)
    out_sorted = _grouped_ffn(x_sorted, W1, b1, W2, b2, starts, ends)
    out = _gather_rows(out_sorted, dest)
    return out.reshape(B, T, D)
